# baseline (speedup 1.0000x reference)
import functools

import jax
import jax.numpy as jnp
from jax.experimental import pallas as pl
from jax.experimental.pallas import tpu as pltpu

M, B, DIN, DOUT = 8, 1024, 1024, 1024
BB = 512


def _mm_kernel(x_ref, w_ref, b_ref, o_ref):
    x = x_ref[0]
    w = w_ref[0]
    acc = jax.lax.dot_general(
        x, w, (((1,), (0,)), ((), ())),
        preferred_element_type=jnp.float32,
    )
    o_ref[0] = acc + b_ref[0]


@functools.partial(jax.jit)
def kernel(inputs, W, b):
    grid = (M, B // BB)
    out = pl.pallas_call(
        _mm_kernel,
        grid=grid,
        in_specs=[
            pl.BlockSpec((1, BB, DIN), lambda m, i: (m, i, 0)),
            pl.BlockSpec((1, DIN, DOUT), lambda m, i: (m, 0, 0)),
            pl.BlockSpec((1, 1, DOUT), lambda m, i: (m, 0, 0)),
        ],
        out_specs=pl.BlockSpec((1, BB, DOUT), lambda m, i: (m, i, 0)),
        out_shape=jax.ShapeDtypeStruct((M, B, DOUT), jnp.float32),
        compiler_params=pltpu.CompilerParams(
            dimension_semantics=("arbitrary", "arbitrary"),
        ),
    )(inputs, W, b.reshape(M, 1, DOUT))
    return out


# speedup vs baseline: 1.0019x; 1.0019x over previous
import functools

import jax
import jax.numpy as jnp
from jax.experimental import pallas as pl
from jax.experimental.pallas import tpu as pltpu

M, B, DIN, DOUT = 8, 1024, 1024, 1024
BB = 512


def _mm_kernel(x_ref, w_ref, b_ref, o_ref):
    x = x_ref[0].astype(jnp.bfloat16)
    w = w_ref[0].astype(jnp.bfloat16)
    acc = jax.lax.dot_general(
        x, w, (((1,), (0,)), ((), ())),
        preferred_element_type=jnp.float32,
    )
    o_ref[0] = acc + b_ref[0]


@functools.partial(jax.jit)
def kernel(inputs, W, b):
    grid = (M, B // BB)
    out = pl.pallas_call(
        _mm_kernel,
        grid=grid,
        in_specs=[
            pl.BlockSpec((1, BB, DIN), lambda m, i: (m, i, 0)),
            pl.BlockSpec((1, DIN, DOUT), lambda m, i: (m, 0, 0)),
            pl.BlockSpec((1, 1, DOUT), lambda m, i: (m, 0, 0)),
        ],
        out_specs=pl.BlockSpec((1, BB, DOUT), lambda m, i: (m, i, 0)),
        out_shape=jax.ShapeDtypeStruct((M, B, DOUT), jnp.float32),
        compiler_params=pltpu.CompilerParams(
            dimension_semantics=("arbitrary", "arbitrary"),
        ),
    )(inputs, W, b.reshape(M, 1, DOUT))
    return out


# speedup vs baseline: 1.0025x; 1.0006x over previous
import functools

import jax
import jax.numpy as jnp
from jax.experimental import pallas as pl
from jax.experimental.pallas import tpu as pltpu

M, B, DIN, DOUT = 8, 1024, 1024, 1024
BB = 512


def _mm_kernel(x_ref, w_ref, b_ref, o_ref):
    x = x_ref[0].astype(jnp.bfloat16)
    w = w_ref[0].astype(jnp.bfloat16)
    acc = jax.lax.dot_general(
        x, w, (((1,), (0,)), ((), ())),
        preferred_element_type=jnp.float32,
    )
    o_ref[0] = acc + b_ref[0]


@functools.partial(jax.jit)
def kernel(inputs, W, b):
    grid = (M, B // BB)
    out = pl.pallas_call(
        _mm_kernel,
        grid=grid,
        in_specs=[
            pl.BlockSpec((1, BB, DIN), lambda m, i: (m, i, 0)),
            pl.BlockSpec((1, DIN, DOUT), lambda m, i: (m, 0, 0)),
            pl.BlockSpec((1, 1, DOUT), lambda m, i: (m, 0, 0)),
        ],
        out_specs=pl.BlockSpec((1, BB, DOUT), lambda m, i: (m, i, 0)),
        out_shape=jax.ShapeDtypeStruct((M, B, DOUT), jnp.float32),
        compiler_params=pltpu.CompilerParams(
            dimension_semantics=("parallel", "arbitrary"),
        ),
    )(inputs, W, b.reshape(M, 1, DOUT))
    return out


# speedup vs baseline: 1.2416x; 1.2385x over previous
import functools

import jax
import jax.numpy as jnp
from jax.experimental import pallas as pl
from jax.experimental.pallas import tpu as pltpu

M, B, DIN, DOUT = 8, 1024, 1024, 1024
BB = 1024


def _mm_kernel(x_ref, w_ref, b_ref, o_ref):
    x = x_ref[0].astype(jnp.bfloat16)
    w = w_ref[0].astype(jnp.bfloat16)
    acc = jax.lax.dot_general(
        x, w, (((1,), (0,)), ((), ())),
        preferred_element_type=jnp.float32,
    )
    o_ref[0] = acc + b_ref[0]


@functools.partial(jax.jit)
def kernel(inputs, W, b):
    grid = (M, B // BB)
    out = pl.pallas_call(
        _mm_kernel,
        grid=grid,
        in_specs=[
            pl.BlockSpec((1, BB, DIN), lambda m, i: (m, i, 0)),
            pl.BlockSpec((1, DIN, DOUT), lambda m, i: (m, 0, 0)),
            pl.BlockSpec((1, 1, DOUT), lambda m, i: (m, 0, 0)),
        ],
        out_specs=pl.BlockSpec((1, BB, DOUT), lambda m, i: (m, i, 0)),
        out_shape=jax.ShapeDtypeStruct((M, B, DOUT), jnp.float32),
        compiler_params=pltpu.CompilerParams(
            dimension_semantics=("parallel", "arbitrary"),
        ),
    )(inputs, W, b.reshape(M, 1, DOUT))
    return out
